# baseline (speedup 1.0000x reference)
import functools

import jax
import jax.numpy as jnp
from jax import lax
from jax.experimental import pallas as pl
from jax.experimental.pallas import tpu as pltpu
from jax.experimental.pallas import tpu_sc as plsc

B, N, D = 16, 4096, 256
L = 16
NC, NS = 2, 16
NW = NC * NS
SC_B = 4
TC_B = B - SC_B

_INV_N = 1.0 / N
_INV_NM1 = 1.0 / (N - 1)


def _sqrt16(v):
    i = lax.bitcast_convert_type(v, jnp.int32)
    i = jnp.int32(0x5F3759DF) - (i >> 1)
    y = lax.bitcast_convert_type(i, jnp.float32)
    half_v = v * jnp.float32(0.5)
    for _ in range(3):
        y = y * (jnp.float32(1.5) - half_v * y * y)
    s = v * y
    return jnp.where(v > jnp.float32(0.0), s, jnp.float32(0.0))


def _make_sc_pool(nb):
    w = D * nb // NW
    g = w // L
    per_b = NW // nb
    ch = min(N, 32768 // w)
    nchunk = N // ch

    mesh = plsc.VectorSubcoreMesh(core_axis_name="c", subcore_axis_name="s")

    @functools.partial(
        pl.kernel,
        mesh=mesh,
        compiler_params=pltpu.CompilerParams(use_tc_tiling_on_sc=False),
        out_type=jax.ShapeDtypeStruct((nb, 4, D), jnp.float32),
        scratch_types=[
            pltpu.VMEM((2, ch, w), jnp.float32),
            pltpu.VMEM((4, w), jnp.float32),
            pltpu.SemaphoreType.DMA,
            pltpu.SemaphoreType.DMA,
        ],
    )
    def sc_pool(x_hbm, out_hbm, buf, res, sem0, sem1):
        wid = lax.axis_index("s") * NC + lax.axis_index("c")
        b = wid // per_b
        h = (wid % per_b) * w
        sems = (sem0, sem1)

        def _src(c):
            return x_hbm.at[b, pl.ds(c * ch, ch), pl.ds(h, w)]

        pltpu.async_copy(_src(0), buf.at[0], sems[0])

        accs = []
        for _ in range(g):
            accs += [
                jnp.full((L,), -jnp.inf, jnp.float32),
                jnp.full((L,), jnp.inf, jnp.float32),
                jnp.zeros((L,), jnp.float32),
                jnp.zeros((L,), jnp.float32),
            ]
        accs = tuple(accs)

        for c in range(nchunk):
            slot = c % 2
            if c + 1 < nchunk:
                pltpu.async_copy(_src(c + 1), buf.at[1 - slot], sems[1 - slot])
            pltpu.make_async_copy(_src(c), buf.at[slot], sems[slot]).wait()
            vbuf = buf.at[slot]

            def body(r, a, vbuf=vbuf):
                out = []
                for k in range(g):
                    v = vbuf[r, pl.ds(k * L, L)]
                    mx, mn, sm, sq = a[4 * k : 4 * k + 4]
                    out += [jnp.maximum(mx, v), jnp.minimum(mn, v),
                            sm + v, sq + v * v]
                return tuple(out)

            accs = plsc.parallel_loop(0, ch, carry=accs, unroll=8)(body)

        for k in range(g):
            mx, mn, sm, sq = accs[4 * k : 4 * k + 4]
            mean = sm * jnp.float32(_INV_N)
            var = (sq - sm * mean) * jnp.float32(_INV_NM1)
            std = _sqrt16(jnp.maximum(var, jnp.float32(0.0)))
            sl = pl.ds(k * L, L)
            res[0, sl] = mx
            res[1, sl] = mn
            res[2, sl] = mean
            res[3, sl] = std

        pltpu.sync_copy(res, out_hbm.at[b, :, pl.ds(h, w)])

    return sc_pool


def _tc_body(x_ref, o_ref):
    xb = x_ref[...]
    mx = jnp.max(xb, axis=0)
    mn = jnp.min(xb, axis=0)
    sm = jnp.sum(xb, axis=0)
    sq = jnp.sum(xb * xb, axis=0)
    mean = sm * jnp.float32(_INV_N)
    var = (sq - sm * mean) * jnp.float32(_INV_NM1)
    std = jnp.sqrt(jnp.maximum(var, jnp.float32(0.0)))
    o_ref[...] = jnp.stack([mx, mn, mean, std], axis=0)


def _make_tc_pool(nb):
    return pl.pallas_call(
        _tc_body,
        grid=(nb,),
        in_specs=[pl.BlockSpec((None, N, D), lambda i: (i, 0, 0))],
        out_specs=pl.BlockSpec((None, 4, D), lambda i: (i, 0, 0)),
        out_shape=jax.ShapeDtypeStruct((nb, 4, D), jnp.float32),
    )


_sc_pool = _make_sc_pool(SC_B) if SC_B else None
_tc_pool = _make_tc_pool(TC_B) if TC_B else None


def kernel(x):
    parts = []
    if TC_B:
        parts.append(_tc_pool(x[:TC_B]))
    if SC_B:
        parts.append(_sc_pool(x[TC_B:]))
    return parts[0] if len(parts) == 1 else jnp.concatenate(parts, axis=0)


# speedup vs baseline: 1.7855x; 1.7855x over previous
import functools

import jax
import jax.numpy as jnp
from jax import lax
from jax.experimental import pallas as pl
from jax.experimental.pallas import tpu as pltpu
from jax.experimental.pallas import tpu_sc as plsc

B, N, D = 16, 4096, 256
L = 16
NC, NS = 2, 16
NW = NC * NS
SC_B = 16
TC_B = B - SC_B

_INV_N = 1.0 / N
_INV_NM1 = 1.0 / (N - 1)


def _sqrt16(v):
    i = lax.bitcast_convert_type(v, jnp.int32)
    i = jnp.int32(0x5F3759DF) - (i >> 1)
    y = lax.bitcast_convert_type(i, jnp.float32)
    half_v = v * jnp.float32(0.5)
    for _ in range(3):
        y = y * (jnp.float32(1.5) - half_v * y * y)
    s = v * y
    return jnp.where(v > jnp.float32(0.0), s, jnp.float32(0.0))


def _make_sc_pool(nb):
    w = D * nb // NW
    g = w // L
    per_b = NW // nb
    ch = min(N, 32768 // w)
    nchunk = N // ch

    mesh = plsc.VectorSubcoreMesh(core_axis_name="c", subcore_axis_name="s")

    @functools.partial(
        pl.kernel,
        mesh=mesh,
        out_type=jax.ShapeDtypeStruct((nb, 4, D), jnp.float32),
        scratch_types=[
            pltpu.VMEM((2, ch, w), jnp.float32),
            pltpu.VMEM((4, w), jnp.float32),
            pltpu.SemaphoreType.DMA,
            pltpu.SemaphoreType.DMA,
        ],
    )
    def sc_pool(x_hbm, out_hbm, buf, res, sem0, sem1):
        wid = lax.axis_index("s") * NC + lax.axis_index("c")
        b = wid // per_b
        h = (wid % per_b) * w
        sems = (sem0, sem1)

        def _src(c):
            return x_hbm.at[b, pl.ds(c * ch, ch), pl.ds(h, w)]

        pltpu.async_copy(_src(0), buf.at[0], sems[0])

        accs = []
        for _ in range(g):
            accs += [
                jnp.full((L,), -jnp.inf, jnp.float32),
                jnp.full((L,), jnp.inf, jnp.float32),
                jnp.zeros((L,), jnp.float32),
                jnp.zeros((L,), jnp.float32),
            ]
        accs = tuple(accs)

        for c in range(nchunk):
            slot = c % 2
            if c + 1 < nchunk:
                pltpu.async_copy(_src(c + 1), buf.at[1 - slot], sems[1 - slot])
            pltpu.make_async_copy(_src(c), buf.at[slot], sems[slot]).wait()
            vbuf = buf.at[slot]

            def body(r, a, vbuf=vbuf):
                out = []
                for k in range(g):
                    v = vbuf[r, pl.ds(k * L, L)]
                    mx, mn, sm, sq = a[4 * k : 4 * k + 4]
                    out += [jnp.maximum(mx, v), jnp.minimum(mn, v),
                            sm + v, sq + v * v]
                return tuple(out)

            accs = plsc.parallel_loop(0, ch, carry=accs, unroll=8)(body)

        for k in range(g):
            mx, mn, sm, sq = accs[4 * k : 4 * k + 4]
            mean = sm * jnp.float32(_INV_N)
            var = (sq - sm * mean) * jnp.float32(_INV_NM1)
            std = _sqrt16(jnp.maximum(var, jnp.float32(0.0)))
            sl = pl.ds(k * L, L)
            res[0, sl] = mx
            res[1, sl] = mn
            res[2, sl] = mean
            res[3, sl] = std

        pltpu.sync_copy(res, out_hbm.at[b, :, pl.ds(h, w)])

    return sc_pool


def _tc_body(x_ref, o_ref):
    xb = x_ref[...]
    mx = jnp.max(xb, axis=0)
    mn = jnp.min(xb, axis=0)
    sm = jnp.sum(xb, axis=0)
    sq = jnp.sum(xb * xb, axis=0)
    mean = sm * jnp.float32(_INV_N)
    var = (sq - sm * mean) * jnp.float32(_INV_NM1)
    std = jnp.sqrt(jnp.maximum(var, jnp.float32(0.0)))
    o_ref[...] = jnp.stack([mx, mn, mean, std], axis=0)


def _make_tc_pool(nb):
    return pl.pallas_call(
        _tc_body,
        grid=(nb,),
        in_specs=[pl.BlockSpec((None, N, D), lambda i: (i, 0, 0))],
        out_specs=pl.BlockSpec((None, 4, D), lambda i: (i, 0, 0)),
        out_shape=jax.ShapeDtypeStruct((nb, 4, D), jnp.float32),
    )


_sc_pool = _make_sc_pool(SC_B) if SC_B else None
_tc_pool = _make_tc_pool(TC_B) if TC_B else None


def kernel(x):
    parts = []
    if TC_B:
        parts.append(_tc_pool(x[:TC_B]))
    if SC_B:
        parts.append(_sc_pool(x[TC_B:]))
    return parts[0] if len(parts) == 1 else jnp.concatenate(parts, axis=0)


# speedup vs baseline: 2.4569x; 1.3760x over previous
import functools

import jax
import jax.numpy as jnp
from jax import lax
from jax.experimental import pallas as pl
from jax.experimental.pallas import tpu as pltpu
from jax.experimental.pallas import tpu_sc as plsc

B, N, D = 16, 4096, 256
L = 16
NC, NS = 2, 16
NW = NC * NS
HW = 128
G = HW // L
SC_B = 4
TC_B = B - SC_B

_INV_N = 1.0 / N
_INV_NM1 = 1.0 / (N - 1)


def _sqrt16(v):
    i = lax.bitcast_convert_type(v, jnp.int32)
    i = jnp.int32(0x5F3759DF) - (i >> 1)
    y = lax.bitcast_convert_type(i, jnp.float32)
    half_v = v * jnp.float32(0.5)
    for _ in range(3):
        y = y * (jnp.float32(1.5) - half_v * y * y)
    s = v * y
    return jnp.where(v > jnp.float32(0.0), s, jnp.float32(0.0))


def _make_sc_pool(nb, first_b):
    rq = NS * NC // (2 * nb)
    rpw = N // rq
    ch = min(rpw, 256)
    nchunk = rpw // ch
    nb_core = nb // NC

    mesh = plsc.VectorSubcoreMesh(core_axis_name="c", subcore_axis_name="s")

    @functools.partial(
        pl.kernel,
        mesh=mesh,
        out_type=jax.ShapeDtypeStruct((nb, 4, D), jnp.float32),
        scratch_types=[
            pltpu.VMEM((2, ch, HW), jnp.float32),
            pltpu.VMEM((4, HW), jnp.float32),
            pltpu.VMEM((rq, 4, HW), jnp.float32),
            pltpu.VMEM_SHARED((NS, 4, HW), jnp.float32),
            pltpu.SemaphoreType.DMA,
            pltpu.SemaphoreType.DMA,
        ],
    )
    def sc_pool(x_hbm, out_hbm, buf, res, comb, shared, sem0, sem1):
        c = lax.axis_index("c")
        s = lax.axis_index("s")
        b_loc = s // (2 * rq)
        rem = s % (2 * rq)
        half = rem // rq
        q = rem % rq
        b = first_b + c * nb_core + b_loc
        h = half * HW
        r0 = q * rpw
        sems = (sem0, sem1)

        def _src(ci):
            return x_hbm.at[b, pl.ds(r0 + ci * ch, ch), pl.ds(h, HW)]

        pltpu.async_copy(_src(0), buf.at[0], sems[0])

        accs = []
        for _ in range(G):
            accs += [
                jnp.full((L,), -jnp.inf, jnp.float32),
                jnp.full((L,), jnp.inf, jnp.float32),
                jnp.zeros((L,), jnp.float32),
                jnp.zeros((L,), jnp.float32),
            ]
        accs = tuple(accs)

        for ci in range(nchunk):
            slot = ci % 2
            if ci + 1 < nchunk:
                pltpu.async_copy(_src(ci + 1), buf.at[1 - slot], sems[1 - slot])
            pltpu.make_async_copy(_src(ci), buf.at[slot], sems[slot]).wait()
            vbuf = buf.at[slot]

            def body(r, a, vbuf=vbuf):
                out = []
                for g in range(G):
                    v = vbuf[r, pl.ds(g * L, L)]
                    mx, mn, sm, sq = a[4 * g : 4 * g + 4]
                    out += [jnp.maximum(mx, v), jnp.minimum(mn, v),
                            sm + v, sq + v * v]
                return tuple(out)

            accs = plsc.parallel_loop(0, ch, carry=accs, unroll=8)(body)

        for g in range(G):
            mx, mn, sm, sq = accs[4 * g : 4 * g + 4]
            sl = pl.ds(g * L, L)
            res[0, sl] = mx
            res[1, sl] = mn
            res[2, sl] = sm
            res[3, sl] = sq
        pltpu.sync_copy(res, shared.at[s])
        plsc.subcore_barrier()

        @pl.when(q == 0)
        def _():
            pltpu.sync_copy(shared.at[pl.ds(s, rq)], comb)
            for g in range(G):
                sl = pl.ds(g * L, L)
                mx = comb[0, 0, sl]
                mn = comb[0, 1, sl]
                sm = comb[0, 2, sl]
                sq = comb[0, 3, sl]
                for j in range(1, rq):
                    mx = jnp.maximum(mx, comb[j, 0, sl])
                    mn = jnp.minimum(mn, comb[j, 1, sl])
                    sm = sm + comb[j, 2, sl]
                    sq = sq + comb[j, 3, sl]
                mean = sm * jnp.float32(_INV_N)
                var = (sq - sm * mean) * jnp.float32(_INV_NM1)
                std = _sqrt16(jnp.maximum(var, jnp.float32(0.0)))
                res[0, sl] = mx
                res[1, sl] = mn
                res[2, sl] = mean
                res[3, sl] = std
            pltpu.sync_copy(res, out_hbm.at[b - first_b, :, pl.ds(h, HW)])

    return sc_pool


def _tc_body(x_ref, o_ref):
    xb = x_ref[...]
    mx = jnp.max(xb, axis=0)
    mn = jnp.min(xb, axis=0)
    sm = jnp.sum(xb, axis=0)
    sq = jnp.sum(xb * xb, axis=0)
    mean = sm * jnp.float32(_INV_N)
    var = (sq - sm * mean) * jnp.float32(_INV_NM1)
    std = jnp.sqrt(jnp.maximum(var, jnp.float32(0.0)))
    o_ref[...] = jnp.stack([mx, mn, mean, std], axis=0)


def _make_tc_pool(nb):
    return pl.pallas_call(
        _tc_body,
        grid=(nb,),
        in_specs=[pl.BlockSpec((None, N, D), lambda i: (i, 0, 0))],
        out_specs=pl.BlockSpec((None, 4, D), lambda i: (i, 0, 0)),
        out_shape=jax.ShapeDtypeStruct((nb, 4, D), jnp.float32),
    )


_sc_pool = _make_sc_pool(SC_B, TC_B) if SC_B else None
_tc_pool = _make_tc_pool(TC_B) if TC_B else None


def kernel(x):
    parts = []
    if TC_B:
        parts.append(_tc_pool(x))
    if SC_B:
        parts.append(_sc_pool(x))
    return parts[0] if len(parts) == 1 else jnp.concatenate(parts, axis=0)


# speedup vs baseline: 2.6960x; 1.0973x over previous
import functools

import jax
import jax.numpy as jnp
from jax import lax
from jax.experimental import pallas as pl
from jax.experimental.pallas import tpu as pltpu
from jax.experimental.pallas import tpu_sc as plsc

B, N, D = 16, 4096, 256
L = 16
NC, NS = 2, 16
NW = NC * NS
HW = 128
G = HW // L
SC_B = 4
TC_B = B - SC_B

_INV_N = 1.0 / N
_INV_NM1 = 1.0 / (N - 1)


def _sqrt16(v):
    i = lax.bitcast_convert_type(v, jnp.int32)
    i = jnp.int32(0x5F3759DF) - (i >> 1)
    y = lax.bitcast_convert_type(i, jnp.float32)
    half_v = v * jnp.float32(0.5)
    for _ in range(3):
        y = y * (jnp.float32(1.5) - half_v * y * y)
    s = v * y
    return jnp.where(v > jnp.float32(0.0), s, jnp.float32(0.0))


def _make_sc_pool(nb, first_b):
    rq = NS * NC // (2 * nb)
    rpw = N // rq
    ch = min(rpw, 256)
    nchunk = rpw // ch
    nb_core = nb // NC

    mesh = plsc.VectorSubcoreMesh(core_axis_name="c", subcore_axis_name="s")

    @functools.partial(
        pl.kernel,
        mesh=mesh,
        out_type=jax.ShapeDtypeStruct((nb, 4, D), jnp.float32),
        scratch_types=[
            pltpu.VMEM((2, ch, HW), jnp.float32),
            pltpu.VMEM((4, HW), jnp.float32),
            pltpu.VMEM((rq, 4, HW), jnp.float32),
            pltpu.VMEM_SHARED((NS, 4, HW), jnp.float32),
            pltpu.SemaphoreType.DMA,
            pltpu.SemaphoreType.DMA,
        ],
    )
    def sc_pool(x_hbm, out_hbm, buf, res, comb, shared, sem0, sem1):
        c = lax.axis_index("c")
        s = lax.axis_index("s")
        b_loc = s // (2 * rq)
        rem = s % (2 * rq)
        half = rem // rq
        q = rem % rq
        b = first_b + c * nb_core + b_loc
        h = half * HW
        r0 = q * rpw
        sems = (sem0, sem1)

        def _src(ci):
            return x_hbm.at[b, pl.ds(r0 + ci * ch, ch), pl.ds(h, HW)]

        pltpu.async_copy(_src(0), buf.at[0], sems[0])

        accs = []
        for _ in range(G):
            accs += [
                jnp.full((L,), -jnp.inf, jnp.float32),
                jnp.full((L,), jnp.inf, jnp.float32),
                jnp.zeros((L,), jnp.float32),
                jnp.zeros((L,), jnp.float32),
            ]
        accs = tuple(accs)

        for ci in range(nchunk):
            slot = ci % 2
            if ci + 1 < nchunk:
                pltpu.async_copy(_src(ci + 1), buf.at[1 - slot], sems[1 - slot])
            pltpu.make_async_copy(_src(ci), buf.at[slot], sems[slot]).wait()
            vbuf = buf.at[slot]

            def body(r, a, vbuf=vbuf):
                out = []
                for g in range(G):
                    v = vbuf[r, pl.ds(g * L, L)]
                    mx, mn, sm, sq = a[4 * g : 4 * g + 4]
                    out += [jnp.maximum(mx, v), jnp.minimum(mn, v),
                            sm + v, sq + v * v]
                return tuple(out)

            accs = plsc.parallel_loop(0, ch, carry=accs, unroll=8)(body)

        for g in range(G):
            mx, mn, sm, sq = accs[4 * g : 4 * g + 4]
            sl = pl.ds(g * L, L)
            res[0, sl] = mx
            res[1, sl] = mn
            res[2, sl] = sm
            res[3, sl] = sq
        pltpu.sync_copy(res, shared.at[s])
        plsc.subcore_barrier()

        @pl.when(q == 0)
        def _():
            pltpu.sync_copy(shared.at[pl.ds(s, rq)], comb)
            for g in range(G):
                sl = pl.ds(g * L, L)
                mx = comb[0, 0, sl]
                mn = comb[0, 1, sl]
                sm = comb[0, 2, sl]
                sq = comb[0, 3, sl]
                for j in range(1, rq):
                    mx = jnp.maximum(mx, comb[j, 0, sl])
                    mn = jnp.minimum(mn, comb[j, 1, sl])
                    sm = sm + comb[j, 2, sl]
                    sq = sq + comb[j, 3, sl]
                mean = sm * jnp.float32(_INV_N)
                var = (sq - sm * mean) * jnp.float32(_INV_NM1)
                std = _sqrt16(jnp.maximum(var, jnp.float32(0.0)))
                res[0, sl] = mx
                res[1, sl] = mn
                res[2, sl] = mean
                res[3, sl] = std
            pltpu.sync_copy(res, out_hbm.at[b - first_b, :, pl.ds(h, HW)])

    return sc_pool


def _tc_body(x_ref, o_ref):
    xb = x_ref[...]
    mx = jnp.max(xb, axis=0)
    mn = jnp.min(xb, axis=0)
    ones = jnp.ones((1, N), jnp.float32)
    sm = jnp.dot(ones, xb, preferred_element_type=jnp.float32)[0]
    sq = jnp.dot(ones, xb * xb, preferred_element_type=jnp.float32)[0]
    mean = sm * jnp.float32(_INV_N)
    var = (sq - sm * mean) * jnp.float32(_INV_NM1)
    std = jnp.sqrt(jnp.maximum(var, jnp.float32(0.0)))
    o_ref[...] = jnp.stack([mx, mn, mean, std], axis=0)


def _make_tc_pool(nb):
    return pl.pallas_call(
        _tc_body,
        grid=(nb,),
        in_specs=[pl.BlockSpec((None, N, D), lambda i: (i, 0, 0))],
        out_specs=pl.BlockSpec((None, 4, D), lambda i: (i, 0, 0)),
        out_shape=jax.ShapeDtypeStruct((nb, 4, D), jnp.float32),
    )


_sc_pool = _make_sc_pool(SC_B, TC_B) if SC_B else None
_tc_pool = _make_tc_pool(TC_B) if TC_B else None


def kernel(x):
    parts = []
    if TC_B:
        parts.append(_tc_pool(x))
    if SC_B:
        parts.append(_sc_pool(x))
    return parts[0] if len(parts) == 1 else jnp.concatenate(parts, axis=0)
